# baseline (speedup 1.0000x reference)
import functools

import jax
import jax.numpy as jnp
from jax import lax
from jax.experimental import pallas as pl
from jax.experimental.pallas import tpu as pltpu
from jax.experimental.pallas import tpu_sc as plsc

N_STATIONS = 100000
EMBED_DIM = 32
BATCH = 16384

_NC = 2
_NS = 16
_NW = _NC * _NS
_CHUNK = 4096
_NCHUNK = BATCH // _CHUNK

_mesh = plsc.VectorSubcoreMesh(core_axis_name="c", subcore_axis_name="s")


@functools.partial(
    pl.kernel,
    mesh=_mesh,
    out_type=jax.ShapeDtypeStruct((EMBED_DIM, BATCH), jnp.float32),
    scratch_types=[
        pltpu.VMEM((BATCH,), jnp.int32),
        pltpu.VMEM((N_STATIONS,), jnp.float32),
        pltpu.VMEM((_CHUNK,), jnp.float32),
        pltpu.VMEM((_CHUNK,), jnp.float32),
        pltpu.VMEM_SHARED((BATCH,), jnp.int32),
        pltpu.SemaphoreType.DMA,
        pltpu.SemaphoreType.DMA,
    ],
    compiler_params=pltpu.CompilerParams(needs_layout_passes=False),
)
def _gather_kernel(
    idx_hbm, table_hbm, out_hbm, idx_v, row_v, out_a, out_b, idx_sh, sem_in, sem_out
):
    sid = lax.axis_index("s")
    wid = sid * _NC + lax.axis_index("c")

    row_cp = pltpu.async_copy(table_hbm.at[wid], row_v, sem_in)

    @pl.when(sid == 0)
    def _():
        pltpu.sync_copy(idx_hbm, idx_sh)

    plsc.subcore_barrier()
    pltpu.sync_copy(idx_sh, idx_v)
    row_cp.wait()

    bufs = (out_a, out_b)
    handles = [None, None]
    for c in range(_NCHUNK):
        b = c % 2
        if handles[b] is not None:
            handles[b].wait()

        @plsc.parallel_loop(0, _CHUNK, step=16, unroll=8)
        def _g(u, c=c, buf=bufs[b]):
            iv = idx_v[pl.ds(c * _CHUNK + u, 16)]
            buf[pl.ds(u, 16)] = plsc.load_gather(row_v, [iv])

        handles[b] = pltpu.async_copy(
            bufs[b], out_hbm.at[wid, pl.ds(c * _CHUNK, _CHUNK)], sem_out
        )
    handles[0].wait()
    handles[1].wait()


def kernel(station_ids, weight):
    out_t = _gather_kernel(station_ids.astype(jnp.int32), weight.T)
    return out_t.T
